# baseline (speedup 1.0000x reference)
import jax
import jax.numpy as jnp
from jax.experimental import pallas as pl
from jax.experimental.pallas import tpu as pltpu

N = 4096
IN_CH = 512
HID = 64
BLOCK = 1024


def _body(fts_ref, adj_ref, b_ref, a_ref, out_ref):
    out = jnp.dot(adj_ref[...], fts_ref[...], preferred_element_type=jnp.float32)
    out = out + b_ref[...]
    a = a_ref[0, 0]
    out_ref[...] = jnp.where(out > 0.0, out, a * out)


def kernel(seq, adj, W, bias, prelu_a):
    fts = seq @ W.T
    b2 = bias.reshape(1, HID)
    a2 = jnp.asarray(prelu_a, jnp.float32).reshape(1, 1)

    grid = (N // BLOCK,)
    return pl.pallas_call(
        _body,
        grid=grid,
        in_specs=[
            pl.BlockSpec((N, HID), lambda i: (0, 0)),
            pl.BlockSpec((BLOCK, N), lambda i: (i, 0)),
            pl.BlockSpec((1, HID), lambda i: (0, 0)),
            pl.BlockSpec(memory_space=pltpu.SMEM),
        ],
        out_specs=pl.BlockSpec((BLOCK, HID), lambda i: (i, 0)),
        out_shape=jax.ShapeDtypeStruct((N, HID), jnp.float32),
    )(fts, adj, b2, a2)


# speedup vs baseline: 1.0257x; 1.0257x over previous
import jax
import jax.numpy as jnp
from jax.experimental import pallas as pl
from jax.experimental.pallas import tpu as pltpu

N = 4096
IN_CH = 512
HID = 64
BLOCK = 512
NSTEPS = N // BLOCK
NBUF = 4


def _body(seq_hbm, adj_hbm, wt_ref, b_ref, a_ref, out_ref,
          seq_buf, fts_ref, *rest):
    bufs = rest[:NBUF]
    seq_sem = rest[NBUF]
    sems = rest[NBUF + 1:]

    def adj_copy(chunk, slot):
        return pltpu.make_async_copy(
            adj_hbm.at[pl.ds(chunk * BLOCK, BLOCK), :], bufs[slot], sems[slot]
        )

    adj_copy(0, 0).start()
    seq_cp = pltpu.make_async_copy(seq_hbm, seq_buf, seq_sem)
    seq_cp.start()
    for s in range(1, NBUF):
        adj_copy(s, s).start()

    seq_cp.wait()
    fts_ref[...] = jnp.dot(
        seq_buf[...], wt_ref[...], preferred_element_type=jnp.float32
    )

    a = a_ref[0, 0]
    for i in range(NSTEPS):
        slot = i % NBUF
        adj_copy(i, slot).wait()
        out = jnp.dot(
            bufs[slot][...], fts_ref[...], preferred_element_type=jnp.float32
        )
        out = out + b_ref[...]
        out_ref[pl.ds(i * BLOCK, BLOCK), :] = jnp.where(out > 0.0, out, a * out)
        nxt = i + NBUF
        if nxt < NSTEPS:
            adj_copy(nxt, slot).start()


def kernel(seq, adj, W, bias, prelu_a):
    wt = W.T
    b2 = bias.reshape(1, HID)
    a2 = jnp.asarray(prelu_a, jnp.float32).reshape(1, 1)

    scratch = (
        [pltpu.VMEM((N, IN_CH), jnp.float32),
         pltpu.VMEM((N, HID), jnp.float32)]
        + [pltpu.VMEM((BLOCK, N), jnp.float32) for _ in range(NBUF)]
        + [pltpu.SemaphoreType.DMA]
        + [pltpu.SemaphoreType.DMA for _ in range(NBUF)]
    )
    return pl.pallas_call(
        _body,
        in_specs=[
            pl.BlockSpec(memory_space=pltpu.MemorySpace.HBM),
            pl.BlockSpec(memory_space=pltpu.MemorySpace.HBM),
            pl.BlockSpec((IN_CH, HID), lambda: (0, 0)),
            pl.BlockSpec((1, HID), lambda: (0, 0)),
            pl.BlockSpec(memory_space=pltpu.SMEM),
        ],
        out_specs=pl.BlockSpec((N, HID), lambda: (0, 0)),
        out_shape=jax.ShapeDtypeStruct((N, HID), jnp.float32),
        scratch_shapes=scratch,
    )(seq, adj, wt, b2, a2)
